# baseline (speedup 1.0000x reference)
import functools

import jax
import jax.numpy as jnp
from jax import lax
from jax.experimental import pallas as pl
from jax.experimental.pallas import tpu as pltpu
from jax.experimental.pallas import tpu_sc as plsc

N = 10000
S = 5
NT = N * S
D = 128
H = 64
HH = H // 2
E = 320000
E5 = S * E
K = 400
CHUNKS = 250
PAIRS = CHUNKS // 2
ACC_ROWS = NT
DEG_TRASH = N
DEG_ROWS = 10112
DEG_EPAD = 327680
_Q = 3128
_QLAST = NT - 15 * _Q

_sc_mesh = plsc.VectorSubcoreMesh(core_axis_name="c", subcore_axis_name="s")


@functools.partial(
    pl.kernel,
    out_type=jax.ShapeDtypeStruct((2, DEG_ROWS, 16), jnp.float32),
    mesh=_sc_mesh,
    compiler_params=pltpu.CompilerParams(use_tc_tiling_on_sc=False),
    scratch_types=[
        pltpu.VMEM((1000,), jnp.int32),
        pltpu.VMEM((1000, 16), jnp.float32),
        pltpu.VMEM_SHARED((DEG_ROWS, 16), jnp.float32),
    ],
)
def _sc_degree(dst1d, zeros_c, ones_c, deg_out, idx_v, ones_v, tbl):
    c = lax.axis_index("c")
    t = lax.axis_index("s")
    pltpu.sync_copy(zeros_c, tbl.at[pl.ds(t * 632, 632)])
    pltpu.sync_copy(ones_c, ones_v)
    plsc.subcore_barrier()

    def body(g, carry):
        base = (c * 16 + t) * 10000 + g * 1000
        pltpu.sync_copy(dst1d.at[pl.ds(base, 1000)], idx_v)
        pltpu.sync_copy(ones_v, tbl.at[idx_v], add=True)
        return carry

    lax.fori_loop(0, 10, body, 0)
    plsc.subcore_barrier()
    pltpu.sync_copy(tbl.at[pl.ds(t * 632, 632)],
                    deg_out.at[c, pl.ds(t * 632, 632)])


@functools.partial(
    pl.kernel,
    out_type=jax.ShapeDtypeStruct((NT, 128), jnp.float32),
    mesh=_sc_mesh,
    compiler_params=pltpu.CompilerParams(use_tc_tiling_on_sc=False),
    scratch_types=[
        pltpu.VMEM_SHARED((ACC_ROWS, HH), jnp.float32),
        pltpu.SemaphoreType.DMA,
        pltpu.SemaphoreType.DMA,
        pltpu.SemaphoreType.DMA,
        pltpu.SemaphoreType.DMA,
    ],
)
def _sc_scatter(hp0, hp1, comb, aggr, acc,
                g0, g1, c0, c1):
    c = lax.axis_index("c")
    t = lax.axis_index("s")

    def _init(hp):
        @pl.when(t < 15)
        def _():
            pltpu.sync_copy(hp.at[pl.ds(t * _Q, _Q)],
                            acc.at[pl.ds(t * _Q, _Q)])

        @pl.when(t == 15)
        def _():
            pltpu.sync_copy(hp.at[pl.ds(15 * _Q, _QLAST)],
                            acc.at[pl.ds(15 * _Q, _QLAST)])

    @pl.when(c == 0)
    def _():
        _init(hp0)

    @pl.when(c == 1)
    def _():
        _init(hp1)

    plsc.subcore_barrier()

    def _gather(s, r, sem):
        @pl.when(c == 0)
        def _():
            pltpu.async_copy(hp0.at[s], r, sem)

        @pl.when(c == 1)
        def _():
            pltpu.async_copy(hp1.at[s], r, sem)

    def scoped(ca, cb, r0, r1):
        base0 = t * CHUNKS * 2 * K

        pltpu.sync_copy(comb.at[pl.ds(base0, 4 * K)], ca)
        _gather(ca.at[pl.ds(0, K)], r0, g0)
        _gather(ca.at[pl.ds(2 * K, K)], r1, g1)

        def _step(p, cur, nxt):
            d0 = cur.at[pl.ds(K, K)]
            d1 = cur.at[pl.ds(3 * K, K)]
            pltpu.make_async_copy(hp0.at[cur.at[pl.ds(0, K)]], r0, g0).wait()
            pltpu.async_copy(r0, acc.at[d0], c0, add=True)
            pltpu.make_async_copy(hp0.at[cur.at[pl.ds(2 * K, K)]], r1,
                                  g1).wait()
            pltpu.async_copy(r1, acc.at[d1], c1, add=True)

            @pl.when(p < PAIRS - 1)
            def _():
                base = base0 + (2 * p + 2) * 2 * K
                pltpu.sync_copy(comb.at[pl.ds(base, 4 * K)], nxt)
                pltpu.make_async_copy(r0, acc.at[d0], c0).wait()
                _gather(nxt.at[pl.ds(0, K)], r0, g0)
                pltpu.make_async_copy(r1, acc.at[d1], c1).wait()
                _gather(nxt.at[pl.ds(2 * K, K)], r1, g1)

            @pl.when(p == PAIRS - 1)
            def _():
                pltpu.make_async_copy(r0, acc.at[d0], c0).wait()
                pltpu.make_async_copy(r1, acc.at[d1], c1).wait()

        def body(p, carry):
            @pl.when(p % 2 == 0)
            def _():
                _step(p, ca, cb)

            @pl.when(p % 2 == 1)
            def _():
                _step(p, cb, ca)

            return carry

        lax.fori_loop(0, PAIRS, body, 0)

    pl.run_scoped(
        scoped,
        pltpu.VMEM((4 * K,), jnp.int32),
        pltpu.VMEM((4 * K,), jnp.int32),
        pltpu.VMEM((K, HH), jnp.float32),
        pltpu.VMEM((K, HH), jnp.float32),
    )
    plsc.subcore_barrier()

    @pl.when(t < 15)
    def _():
        pltpu.sync_copy(acc.at[pl.ds(t * _Q, _Q)],
                        aggr.at[pl.ds(t * _Q, _Q), pl.ds(c * HH, HH)])

    @pl.when(t == 15)
    def _():
        pltpu.sync_copy(acc.at[pl.ds(15 * _Q, _QLAST)],
                        aggr.at[pl.ds(15 * _Q, _QLAST), pl.ds(c * HH, HH)])


def _tc_mm_body(bd_b, w5_b, attw_b, attb_b, h_b):
    m = lax.dot_general(w5_b[...], attw_b[...], (((1,), (0,)), ((), ())),
                        preferred_element_type=jnp.float32)
    cvec = lax.dot_general(attb_b[...], w5_b[...], (((1,), (1,)), ((), ())),
                           preferred_element_type=jnp.float32)
    h_b[...] = lax.dot_general(bd_b[...], m, (((1,), (1,)), ((), ())),
                               precision=lax.Precision.HIGHEST,
                               preferred_element_type=jnp.float32) + cvec


_tc_mm = pl.pallas_call(
    _tc_mm_body,
    grid=(25,),
    in_specs=[
        pl.BlockSpec((2000, D), lambda i: (i, 0)),
        pl.BlockSpec((H, H), lambda i: (0, 0)),
        pl.BlockSpec((H, D), lambda i: (0, 0)),
        pl.BlockSpec((1, H), lambda i: (0, 0)),
    ],
    out_specs=pl.BlockSpec((2000, H), lambda i: (i, 0)),
    out_shape=jax.ShapeDtypeStruct((NT, H), jnp.float32),
)


def _tc_scale_body(h_b, dg0_b, dg1_b, hp0_b, hp1_b):
    dinv = lax.rsqrt(dg0_b[0, :, 0:1] + dg1_b[0, :, 0:1] + 1.0)
    hp = h_b[...] * dinv
    hp0_b[...] = hp[:, :HH]
    hp1_b[...] = hp[:, HH:]


_tc_scale = pl.pallas_call(
    _tc_scale_body,
    grid=(25,),
    in_specs=[
        pl.BlockSpec((2000, H), lambda i: (i, 0)),
        pl.BlockSpec((1, 2000, 16), lambda i: (0, i % 5, 0)),
        pl.BlockSpec((1, 2000, 16), lambda i: (1, i % 5, 0)),
    ],
    out_specs=[pl.BlockSpec((2000, HH), lambda i: (i, 0))] * 2,
    out_shape=[jax.ShapeDtypeStruct((NT, HH), jnp.float32)] * 2,
)


def _tc_postattn_body(a_b, dg0_b, dg1_b, b5_b, g_b, bb_b, w1_b, w2_b,
                      ab_b, node_b, nb_b):
    dinv = lax.rsqrt(dg0_b[0, :, 0:1] + dg1_b[0, :, 0:1] + 1.0)
    y = a_b[:, :H] * dinv + b5_b[...]
    mu = jnp.mean(y, axis=1, keepdims=True)
    var = jnp.mean((y - mu) * (y - mu), axis=1, keepdims=True)
    z = (y - mu) * lax.rsqrt(var + 1e-5) * g_b[...] + bb_b[...]
    z3 = z.reshape(400, S, H)
    t0 = z3[:, 0, :]
    node_b[...] = z3[:, 0:1, :]
    base = jnp.sum(t0 * w1_b[...], axis=1, keepdims=True) + ab_b[...]
    nbs = [z3[:, j, :] for j in range(1, S)]
    ls = [base + jnp.sum(nb * w2_b[...], axis=1, keepdims=True) for nb in nbs]
    m = jnp.maximum(jnp.maximum(ls[0], ls[1]), jnp.maximum(ls[2], ls[3]))
    es = [jnp.exp(l - m) for l in ls]
    tot = es[0] + es[1] + es[2] + es[3]
    for j in range(4):
        nb_b[:, j:j + 1, :] = (nbs[j] * (es[j] / tot))[:, None, :]


_tc_postattn = pl.pallas_call(
    _tc_postattn_body,
    grid=(25,),
    in_specs=[
        pl.BlockSpec((2000, 128), lambda i: (i, 0)),
        pl.BlockSpec((1, 2000, 16), lambda i: (0, i % 5, 0)),
        pl.BlockSpec((1, 2000, 16), lambda i: (1, i % 5, 0)),
        pl.BlockSpec((1, H), lambda i: (0, 0)),
        pl.BlockSpec((1, H), lambda i: (0, 0)),
        pl.BlockSpec((1, H), lambda i: (0, 0)),
        pl.BlockSpec((1, H), lambda i: (0, 0)),
        pl.BlockSpec((1, H), lambda i: (0, 0)),
        pl.BlockSpec((1, 1), lambda i: (0, 0)),
    ],
    out_specs=[
        pl.BlockSpec((400, 1, H), lambda i: (i, 0, 0)),
        pl.BlockSpec((400, 4, H), lambda i: (i, 0, 0)),
    ],
    out_shape=[
        jax.ShapeDtypeStruct((N, 1, H), jnp.float32),
        jax.ShapeDtypeStruct((N, 4, H), jnp.float32),
    ],
)


def kernel(batched_data, edge_index, att_W, att_b, gcn_W, gcn_b,
           ln_g, ln_b, attn_W, attn_b):
    bd2 = batched_data.reshape(NT, D)
    src = edge_index[0]
    dst = edge_index[1]
    offs = (jnp.arange(S, dtype=jnp.int32) * N)[:, None]
    src5 = (src[None, :] + offs).reshape(-1)
    dst5 = (dst[None, :] + offs).reshape(-1)
    comb = jnp.stack([src5.reshape(-1, K), dst5.reshape(-1, K)],
                     axis=1).reshape(-1)
    zeros_c = jnp.zeros((632, 16), jnp.float32)
    ones_c = jnp.ones((1000, 16), jnp.float32)

    degp = _sc_degree(dst, zeros_c, ones_c)
    h = _tc_mm(bd2, gcn_W[5], att_W, att_b.reshape(1, H))
    hp0, hp1 = _tc_scale(h, degp, degp)
    aggr = _sc_scatter(hp0, hp1, comb)
    node_tensor, neighbor_tensor = _tc_postattn(
        aggr, degp, degp, gcn_b[5].reshape(1, H), ln_g.reshape(1, H),
        ln_b.reshape(1, H), attn_W[:, :H], attn_W[:, H:],
        attn_b.reshape(1, 1))
    return (node_tensor, neighbor_tensor)


# speedup vs baseline: 1.0424x; 1.0424x over previous
import functools

import jax
import jax.numpy as jnp
from jax import lax
from jax.experimental import pallas as pl
from jax.experimental.pallas import tpu as pltpu
from jax.experimental.pallas import tpu_sc as plsc

N = 10000
S = 5
NT = N * S
D = 128
H = 64
HH = H // 2
E = 320000
E5 = S * E
K = 400
CHUNKS = 250
PAIRS = CHUNKS // 2
ACC_ROWS = NT
DEG_TRASH = N
DEG_ROWS = 10112
DEG_EPAD = 327680
_Q = 3128
_QLAST = NT - 15 * _Q

_sc_mesh = plsc.VectorSubcoreMesh(core_axis_name="c", subcore_axis_name="s")


@functools.partial(
    pl.kernel,
    out_type=jax.ShapeDtypeStruct((2, DEG_ROWS, 16), jnp.float32),
    mesh=_sc_mesh,
    compiler_params=pltpu.CompilerParams(use_tc_tiling_on_sc=False),
    scratch_types=[
        pltpu.VMEM((1000,), jnp.int32),
        pltpu.VMEM((1000, 16), jnp.float32),
        pltpu.VMEM_SHARED((DEG_ROWS, 16), jnp.float32),
    ],
)
def _sc_degree(dst1d, zeros_c, ones_c, deg_out, idx_v, ones_v, tbl):
    c = lax.axis_index("c")
    t = lax.axis_index("s")
    pltpu.sync_copy(zeros_c, tbl.at[pl.ds(t * 632, 632)])
    pltpu.sync_copy(ones_c, ones_v)
    plsc.subcore_barrier()

    def body(g, carry):
        base = (c * 16 + t) * 10000 + g * 1000
        pltpu.sync_copy(dst1d.at[pl.ds(base, 1000)], idx_v)
        pltpu.sync_copy(ones_v, tbl.at[idx_v], add=True)
        return carry

    lax.fori_loop(0, 10, body, 0)
    plsc.subcore_barrier()
    pltpu.sync_copy(tbl.at[pl.ds(t * 632, 632)],
                    deg_out.at[c, pl.ds(t * 632, 632)])


@functools.partial(
    pl.kernel,
    out_type=jax.ShapeDtypeStruct((NT, 128), jnp.float32),
    mesh=_sc_mesh,
    compiler_params=pltpu.CompilerParams(use_tc_tiling_on_sc=False),
    scratch_types=[
        pltpu.VMEM_SHARED((ACC_ROWS, HH), jnp.float32),
        pltpu.SemaphoreType.DMA,
        pltpu.SemaphoreType.DMA,
        pltpu.SemaphoreType.DMA,
        pltpu.SemaphoreType.DMA,
    ],
)
def _sc_scatter(hp0, hp1, comb, aggr, acc,
                g0, g1, c0, c1):
    c = lax.axis_index("c")
    t = lax.axis_index("s")

    def _init(hp):
        @pl.when(t < 15)
        def _():
            pltpu.sync_copy(hp.at[pl.ds(t * _Q, _Q)],
                            acc.at[pl.ds(t * _Q, _Q)])

        @pl.when(t == 15)
        def _():
            pltpu.sync_copy(hp.at[pl.ds(15 * _Q, _QLAST)],
                            acc.at[pl.ds(15 * _Q, _QLAST)])

    @pl.when(c == 0)
    def _():
        _init(hp0)

    @pl.when(c == 1)
    def _():
        _init(hp1)

    plsc.subcore_barrier()

    def _gather(s, r, sem):
        @pl.when(c == 0)
        def _():
            pltpu.async_copy(hp0.at[s], r, sem)

        @pl.when(c == 1)
        def _():
            pltpu.async_copy(hp1.at[s], r, sem)

    def scoped(ca, cb, r0, r1):
        base0 = t * CHUNKS * 2 * K

        pltpu.sync_copy(comb.at[pl.ds(base0, 4 * K)], ca)
        _gather(ca.at[pl.ds(0, K)], r0, g0)
        _gather(ca.at[pl.ds(2 * K, K)], r1, g1)

        def _step(p, cur, nxt):
            d0 = cur.at[pl.ds(K, K)]
            d1 = cur.at[pl.ds(3 * K, K)]
            pltpu.make_async_copy(hp0.at[cur.at[pl.ds(0, K)]], r0, g0).wait()
            pltpu.async_copy(r0, acc.at[d0], c0, add=True)
            pltpu.make_async_copy(hp0.at[cur.at[pl.ds(2 * K, K)]], r1,
                                  g1).wait()
            pltpu.async_copy(r1, acc.at[d1], c1, add=True)

            @pl.when(p < PAIRS - 1)
            def _():
                base = base0 + (2 * p + 2) * 2 * K
                pltpu.sync_copy(comb.at[pl.ds(base, 4 * K)], nxt)
                pltpu.make_async_copy(r0, acc.at[d0], c0).wait()
                _gather(nxt.at[pl.ds(0, K)], r0, g0)
                pltpu.make_async_copy(r1, acc.at[d1], c1).wait()
                _gather(nxt.at[pl.ds(2 * K, K)], r1, g1)

            @pl.when(p == PAIRS - 1)
            def _():
                pltpu.make_async_copy(r0, acc.at[d0], c0).wait()
                pltpu.make_async_copy(r1, acc.at[d1], c1).wait()

        def body(p, carry):
            @pl.when(p % 2 == 0)
            def _():
                _step(p, ca, cb)

            @pl.when(p % 2 == 1)
            def _():
                _step(p, cb, ca)

            return carry

        lax.fori_loop(0, PAIRS, body, 0)

    pl.run_scoped(
        scoped,
        pltpu.VMEM((4 * K,), jnp.int32),
        pltpu.VMEM((4 * K,), jnp.int32),
        pltpu.VMEM((K, HH), jnp.float32),
        pltpu.VMEM((K, HH), jnp.float32),
    )
    plsc.subcore_barrier()

    @pl.when(t < 15)
    def _():
        pltpu.sync_copy(acc.at[pl.ds(t * _Q, _Q)],
                        aggr.at[pl.ds(t * _Q, _Q), pl.ds(c * HH, HH)])

    @pl.when(t == 15)
    def _():
        pltpu.sync_copy(acc.at[pl.ds(15 * _Q, _QLAST)],
                        aggr.at[pl.ds(15 * _Q, _QLAST), pl.ds(c * HH, HH)])


def _tc_hp_body(bd_b, dg0_b, dg1_b, w5_b, attw_b, attb_b, hp0_b, hp1_b):
    m = lax.dot_general(w5_b[...], attw_b[...], (((1,), (0,)), ((), ())),
                        preferred_element_type=jnp.float32)
    cvec = lax.dot_general(attb_b[...], w5_b[...], (((1,), (1,)), ((), ())),
                           preferred_element_type=jnp.float32)
    h = lax.dot_general(bd_b[...], m, (((1,), (1,)), ((), ())),
                        precision=lax.Precision.HIGHEST,
                        preferred_element_type=jnp.float32) + cvec
    dinv = lax.rsqrt(dg0_b[0, :, 0:1] + dg1_b[0, :, 0:1] + 1.0)
    hp = h * dinv
    hp0_b[...] = hp[:, :HH]
    hp1_b[...] = hp[:, HH:]


_tc_hp = pl.pallas_call(
    _tc_hp_body,
    grid=(25,),
    in_specs=[
        pl.BlockSpec((2000, D), lambda i: (i, 0)),
        pl.BlockSpec((1, 2000, 16), lambda i: (0, i % 5, 0)),
        pl.BlockSpec((1, 2000, 16), lambda i: (1, i % 5, 0)),
        pl.BlockSpec((H, H), lambda i: (0, 0)),
        pl.BlockSpec((H, D), lambda i: (0, 0)),
        pl.BlockSpec((1, H), lambda i: (0, 0)),
    ],
    out_specs=[pl.BlockSpec((2000, HH), lambda i: (i, 0))] * 2,
    out_shape=[jax.ShapeDtypeStruct((NT, HH), jnp.float32)] * 2,
)


def _tc_postattn_body(a_b, dg0_b, dg1_b, b5_b, g_b, bb_b, w1_b, w2_b,
                      ab_b, node_b, nb_b):
    dinv = lax.rsqrt(dg0_b[0, :, 0:1] + dg1_b[0, :, 0:1] + 1.0)
    y = a_b[:, :H] * dinv + b5_b[...]
    mu = jnp.mean(y, axis=1, keepdims=True)
    var = jnp.mean((y - mu) * (y - mu), axis=1, keepdims=True)
    z = (y - mu) * lax.rsqrt(var + 1e-5) * g_b[...] + bb_b[...]
    z3 = z.reshape(400, S, H)
    t0 = z3[:, 0, :]
    node_b[...] = z3[:, 0:1, :]
    base = jnp.sum(t0 * w1_b[...], axis=1, keepdims=True) + ab_b[...]
    nbs = [z3[:, j, :] for j in range(1, S)]
    ls = [base + jnp.sum(nb * w2_b[...], axis=1, keepdims=True) for nb in nbs]
    m = jnp.maximum(jnp.maximum(ls[0], ls[1]), jnp.maximum(ls[2], ls[3]))
    es = [jnp.exp(l - m) for l in ls]
    tot = es[0] + es[1] + es[2] + es[3]
    for j in range(4):
        nb_b[:, j:j + 1, :] = (nbs[j] * (es[j] / tot))[:, None, :]


_tc_postattn = pl.pallas_call(
    _tc_postattn_body,
    grid=(25,),
    in_specs=[
        pl.BlockSpec((2000, 128), lambda i: (i, 0)),
        pl.BlockSpec((1, 2000, 16), lambda i: (0, i % 5, 0)),
        pl.BlockSpec((1, 2000, 16), lambda i: (1, i % 5, 0)),
        pl.BlockSpec((1, H), lambda i: (0, 0)),
        pl.BlockSpec((1, H), lambda i: (0, 0)),
        pl.BlockSpec((1, H), lambda i: (0, 0)),
        pl.BlockSpec((1, H), lambda i: (0, 0)),
        pl.BlockSpec((1, H), lambda i: (0, 0)),
        pl.BlockSpec((1, 1), lambda i: (0, 0)),
    ],
    out_specs=[
        pl.BlockSpec((400, 1, H), lambda i: (i, 0, 0)),
        pl.BlockSpec((400, 4, H), lambda i: (i, 0, 0)),
    ],
    out_shape=[
        jax.ShapeDtypeStruct((N, 1, H), jnp.float32),
        jax.ShapeDtypeStruct((N, 4, H), jnp.float32),
    ],
)


def kernel(batched_data, edge_index, att_W, att_b, gcn_W, gcn_b,
           ln_g, ln_b, attn_W, attn_b):
    bd2 = batched_data.reshape(NT, D)
    src = edge_index[0]
    dst = edge_index[1]
    offs = (jnp.arange(S, dtype=jnp.int32) * N)[:, None]
    src5 = (src[None, :] + offs).reshape(-1)
    dst5 = (dst[None, :] + offs).reshape(-1)
    comb = jnp.stack([src5.reshape(-1, K), dst5.reshape(-1, K)],
                     axis=1).reshape(-1)
    zeros_c = jnp.zeros((632, 16), jnp.float32)
    ones_c = jnp.ones((1000, 16), jnp.float32)

    degp = _sc_degree(dst, zeros_c, ones_c)
    hp0, hp1 = _tc_hp(bd2, degp, degp, gcn_W[5], att_W, att_b.reshape(1, H))
    aggr = _sc_scatter(hp0, hp1, comb)
    node_tensor, neighbor_tensor = _tc_postattn(
        aggr, degp, degp, gcn_b[5].reshape(1, H), ln_g.reshape(1, H),
        ln_b.reshape(1, H), attn_W[:, :H], attn_W[:, H:],
        attn_b.reshape(1, 1))
    return (node_tensor, neighbor_tensor)
